# baseline (speedup 1.0000x reference)
import math

import jax
import jax.numpy as jnp
from jax import lax
from jax.experimental import pallas as pl
from jax.experimental.pallas import tpu as pltpu
from jax.experimental.pallas import tpu_sc as plsc

D_MODEL_ = 64
SCALE_ = math.sqrt(D_MODEL_)

NC_ = 2
NS_ = 16
NW_ = NC_ * NS_
LANES_ = 16

CHUNK_ = 512


def _make_lookup(batch, d_model):
    assert d_model % LANES_ == 0
    assert batch % (8 * NW_) == 0
    per_w = batch // NW_
    assert per_w % CHUNK_ == 0
    n_chunks = per_w // CHUNK_

    mesh = plsc.VectorSubcoreMesh(core_axis_name="c", subcore_axis_name="s")

    def body(x_hbm, table_hbm, out_hbm, idx_v, rows_v, sem):
        wid = lax.axis_index("s") * NC_ + lax.axis_index("c")
        w_base = wid * per_w

        def chunk_body(g, _):
            base = w_base + g * CHUNK_
            pltpu.sync_copy(x_hbm.at[pl.ds(base, CHUNK_)], idx_v)
            pltpu.async_copy(table_hbm.at[idx_v], rows_v, sem).wait()

            def scale_row(i, _):
                for j in range(d_model // LANES_):
                    sl = pl.ds(j * LANES_, LANES_)
                    rows_v[i, sl] = rows_v[i, sl] * SCALE_
                return 0

            lax.fori_loop(0, CHUNK_, scale_row, 0)
            pltpu.sync_copy(rows_v, out_hbm.at[pl.ds(base, CHUNK_)])
            return 0

        lax.fori_loop(0, n_chunks, chunk_body, 0)

    return pl.kernel(
        body,
        out_type=jax.ShapeDtypeStruct((batch, d_model), jnp.float32),
        mesh=mesh,
        compiler_params=pltpu.CompilerParams(use_tc_tiling_on_sc=False),
        scratch_types=[
            pltpu.VMEM((CHUNK_,), jnp.int32),
            pltpu.VMEM((CHUNK_, d_model), jnp.float32),
            pltpu.SemaphoreType.DMA,
        ],
    )


def kernel(x, table):
    b0, b1 = x.shape
    d = table.shape[1]
    x_flat = x.reshape(b0 * b1).astype(jnp.int32)
    out = _make_lookup(b0 * b1, d)(x_flat, table)
    return out.reshape(b0, b1, d)


# speedup vs baseline: 1.1356x; 1.1356x over previous
import math

import jax
import jax.numpy as jnp
from jax import lax
from jax.experimental import pallas as pl
from jax.experimental.pallas import tpu as pltpu
from jax.experimental.pallas import tpu_sc as plsc

D_MODEL_ = 64
SCALE_ = math.sqrt(D_MODEL_)

NC_ = 2
NS_ = 16
NW_ = NC_ * NS_
LANES_ = 16

CHUNK_ = 400
UNROLL_ = 4


def _make_lookup(batch, d_model):
    assert d_model % LANES_ == 0
    assert batch % (8 * NW_) == 0
    per_w = batch // NW_
    assert per_w % CHUNK_ == 0
    n_chunks = per_w // CHUNK_
    n2 = n_chunks // 2
    assert n_chunks % 2 == 0 and n2 >= 3
    assert CHUNK_ % UNROLL_ == 0
    nj = d_model // LANES_

    mesh = plsc.VectorSubcoreMesh(core_axis_name="c", subcore_axis_name="s")

    def body(x_hbm, table_hbm, out_hbm, i0, i1, r0, r1, o0, o1, gs0, gs1, os0, os1):
        idx_v = (i0, i1)
        rows_v = (r0, r1)
        outb_v = (o0, o1)
        gsem = (gs0, gs1)
        osem = (os0, os1)

        wid = lax.axis_index("s") * NC_ + lax.axis_index("c")
        w_base = wid * per_w

        def idx_load(g, b):
            pltpu.sync_copy(x_hbm.at[pl.ds(w_base + g * CHUNK_, CHUNK_)], idx_v[b])

        def gather_start(b):
            pltpu.async_copy(table_hbm.at[idx_v[b]], rows_v[b], gsem[b])

        def gather_wait(b):
            pltpu.make_async_copy(table_hbm.at[idx_v[b]], rows_v[b], gsem[b]).wait()

        def scale(b):
            def srow(i, _):
                for u in range(UNROLL_):
                    r = i * UNROLL_ + u
                    for j in range(nj):
                        sl = pl.ds(j * LANES_, LANES_)
                        outb_v[b][r, sl] = rows_v[b][r, sl] * SCALE_
                return 0

            lax.fori_loop(0, CHUNK_ // UNROLL_, srow, 0)

        def out_start(g, b):
            pltpu.async_copy(
                outb_v[b], out_hbm.at[pl.ds(w_base + g * CHUNK_, CHUNK_)], osem[b]
            )

        def out_wait(g, b):
            pltpu.make_async_copy(
                outb_v[b], out_hbm.at[pl.ds(w_base + g * CHUNK_, CHUNK_)], osem[b]
            ).wait()

        for b in (0, 1):
            idx_load(b, b)
            gather_start(b)
        for b in (0, 1):
            gather_wait(b)
            scale(b)
            out_start(b, b)
            idx_load(2 + b, b)
            gather_start(b)

        def steady(i, _):
            for b in (0, 1):
                g = 2 * i + b
                gather_wait(b)
                out_wait(g - 2, b)
                scale(b)
                out_start(g, b)
                idx_load(g + 2, b)
                gather_start(b)
            return 0

        lax.fori_loop(1, n2 - 1, steady, 0)

        for b in (0, 1):
            g = n_chunks - 2 + b
            gather_wait(b)
            out_wait(g - 2, b)
            scale(b)
            out_start(g, b)
        for b in (0, 1):
            out_wait(n_chunks - 2 + b, b)

    return pl.kernel(
        body,
        out_type=jax.ShapeDtypeStruct((batch, d_model), jnp.float32),
        mesh=mesh,
        compiler_params=pltpu.CompilerParams(use_tc_tiling_on_sc=False),
        scratch_types=[
            pltpu.VMEM((CHUNK_,), jnp.int32),
            pltpu.VMEM((CHUNK_,), jnp.int32),
            pltpu.VMEM((CHUNK_, d_model), jnp.float32),
            pltpu.VMEM((CHUNK_, d_model), jnp.float32),
            pltpu.VMEM((CHUNK_, d_model), jnp.float32),
            pltpu.VMEM((CHUNK_, d_model), jnp.float32),
            pltpu.SemaphoreType.DMA,
            pltpu.SemaphoreType.DMA,
            pltpu.SemaphoreType.DMA,
            pltpu.SemaphoreType.DMA,
        ],
    )


def kernel(x, table):
    b0, b1 = x.shape
    d = table.shape[1]
    x_flat = x.reshape(b0 * b1).astype(jnp.int32)
    out = _make_lookup(b0 * b1, d)(x_flat, table)
    return out.reshape(b0, b1, d)
